# baseline (speedup 1.0000x reference)
import functools

import jax
import jax.numpy as jnp
from jax.experimental import pallas as pl
from jax.experimental.pallas import tpu as pltpu

B, N, D = 1024, 256, 128
PROT_DECAY = 0.95
TEMP = 0.5
BB = 8

_bf16 = jnp.bfloat16
_f32 = jnp.float32


def _dot_t(a, b):
    return jax.lax.dot_general(
        a.astype(_bf16), b.astype(_bf16), (((1,), (1,)), ((), ())),
        preferred_element_type=_f32)


def _softmax(x):
    m = jnp.max(x, axis=-1, keepdims=True)
    e = jnp.exp(x - m)
    return e / jnp.sum(e, axis=-1, keepdims=True)


def _argmax_first(x, iota):
    m = jnp.max(x, axis=-1, keepdims=True)
    return jnp.min(jnp.where(x == m, iota, N), axis=-1, keepdims=True)


def _sbmem_kernel(h_ref, wk_ref, wv_ref, prot_ref, usage_ref, age_ref,
                  Wk_ref, bk_ref, Wv_ref, bv_ref,
                  Wg_ref, bg_ref, Wsx_ref, bsx_ref,
                  uk_ref, uv_ref, uprot_ref, ow_ref,
                  ws_ref, mp_ref, bs_ref, owr_ref, pm_ref, ms_ref):
    h = h_ref[...]
    ck = jnp.tanh(_dot_t(h, Wk_ref[...]) + bk_ref[...][None, :])
    cv = jnp.tanh(_dot_t(h, Wv_ref[...]) + bv_ref[...][None, :])

    gates = _dot_t(h, Wg_ref[...]) + bg_ref[...][:, 0][None, :]
    write_strength = jax.nn.sigmoid(gates[:, 0:1])
    g_mg = gates[:, 1:2]
    g_bg = gates[:, 2:3]
    importance = jax.nn.sigmoid(gates[:, 3:4])

    ck_norm = jnp.sqrt(jnp.sum(ck * ck, axis=-1, keepdims=True))
    ncand = ck / jnp.maximum(ck_norm, 1e-6)

    wk = wk_ref[...]
    key_norm = jnp.sqrt(jnp.sum(wk * wk, axis=-1, keepdims=True))
    nkeys = wk / jnp.maximum(key_norm, 1e-6)
    sim_all = jax.lax.dot_general(
        nkeys.reshape(BB * N, D).astype(_bf16),
        ncand.astype(_bf16), (((1,), (1,)), ((), ())),
        preferred_element_type=_f32).reshape(BB, N, BB)
    bsel = jax.lax.broadcasted_iota(jnp.int32, (BB, 1, BB), 2) == \
        jax.lax.broadcasted_iota(jnp.int32, (BB, 1, BB), 0)
    sim = jnp.sum(jnp.where(bsel, sim_all, 0.0), axis=-1)

    wv = wv_ref[...]
    val_norm = jnp.sqrt(jnp.sum(wv * wv, axis=-1))
    norm_occ = jnp.clip(val_norm / (D ** 0.5), 0.0, 1.0)
    sx = jax.lax.dot_general(
        wv.reshape(BB * N, D).astype(_bf16), Wsx_ref[...].astype(_bf16),
        (((1,), (1,)), ((), ())), preferred_element_type=_f32)
    learned_occ = jax.nn.sigmoid(sx[:, 0] + bsx_ref[0, 0]).reshape(BB, N)
    learned_prot = jax.nn.sigmoid(sx[:, 1] + bsx_ref[1, 0]).reshape(BB, N)
    occupancy = jnp.clip(0.5 * learned_occ + 0.5 * norm_occ, 0.0, 1.0)
    eff_prot = jnp.clip(0.4 * learned_prot + 0.6 * prot_ref[...], 0.0, 1.0)
    eff_usage = jnp.clip(0.5 * occupancy + 0.5 * usage_ref[...], 0.0, 1.0)
    eff_age = jnp.clip(age_ref[...], 0.0, 1.0)

    replace_scores = (1.15 * (1.0 - occupancy) + 0.85 * (1.0 - eff_prot)
                      + 0.65 * eff_age + 0.45 * (1.0 - eff_usage)
                      + 0.25 * (1.0 - sim))

    iota = jax.lax.broadcasted_iota(jnp.int32, (BB, N), 1)
    merge_index = _argmax_first(_softmax(sim / TEMP), iota)
    replace_index = _argmax_first(_softmax(replace_scores / TEMP), iota)

    merge_oh = (iota == merge_index).astype(_f32)
    max_sim = jnp.sum(sim * merge_oh, axis=-1, keepdims=True)
    matched_occ = jnp.sum(occupancy * merge_oh, axis=-1, keepdims=True)
    matched_usage = jnp.sum(eff_usage * merge_oh, axis=-1, keepdims=True)
    matched_age = jnp.sum(eff_age * merge_oh, axis=-1, keepdims=True)

    merge_pref = jax.nn.sigmoid(g_mg + 2.4 * max_sim
                                + 1.6 * (matched_occ - 0.5)
                                + 1.0 * (matched_usage - 0.5)
                                - 0.8 * matched_age)
    merge_cand = (max_sim > 0.55) & (matched_occ > 0.35)
    use_merge = (merge_pref >= 0.5) & merge_cand
    target_index = jnp.where(use_merge, merge_index, replace_index)
    target_oh = (iota == target_index).astype(_f32)

    binding = jax.nn.sigmoid(g_bg + 2.2 * max_sim)
    conflict = jnp.clip(1.0 - sim, 0.0, 1.0)
    overwrite = ((0.15 + 0.85 * write_strength) * target_oh
                 * (1.0 - 0.65 * eff_prot * conflict))

    key_mix = jnp.where(use_merge, 0.22 + 0.38 * binding, 0.78 + 0.18 * binding)
    value_mix = jnp.where(use_merge, 0.45 + 0.35 * importance, 0.75 + 0.2 * importance)

    owk = (overwrite * key_mix)[:, :, None]
    owv = (overwrite * value_mix)[:, :, None]
    uk_ref[...] = wk + owk * (ck[:, None, :] - wk)
    uv_ref[...] = wv + owv * (cv[:, None, :] - wv)

    prot_boost = overwrite * (0.5 + 0.5 * importance)
    uprot = jnp.clip(prot_ref[...] * PROT_DECAY + prot_boost, 0.0, 1.0)
    uprot_ref[...] = uprot

    ow_ref[...] = overwrite
    ws_ref[...] = write_strength
    mp_ref[...] = merge_pref
    bs_ref[...] = binding
    owr_ref[...] = jnp.mean(overwrite, axis=-1, keepdims=True)
    pm_ref[...] = jnp.mean(uprot, axis=-1, keepdims=True)
    ms_ref[...] = max_sim


@functools.partial(jax.jit, static_argnames=("interpret",))
def _run(hidden, working_keys, working_values, working_protection,
         working_usage, working_age, Wk, bk, Wv, bv, Wg, bg, Wsx, bsx,
         interpret=False):
    grid = (B // BB,)
    big = lambda i: (i, 0, 0)
    row = lambda i: (i, 0)
    full2 = lambda i: (0, 0)
    out_shapes = [
        jax.ShapeDtypeStruct((B, N, D), jnp.float32),
        jax.ShapeDtypeStruct((B, N, D), jnp.float32),
        jax.ShapeDtypeStruct((B, N), jnp.float32),
        jax.ShapeDtypeStruct((B, N), jnp.float32),
    ] + [jax.ShapeDtypeStruct((B, 1), jnp.float32)] * 6
    in_specs = [
        pl.BlockSpec((BB, D), row),
        pl.BlockSpec((BB, N, D), big),
        pl.BlockSpec((BB, N, D), big),
        pl.BlockSpec((BB, N), row),
        pl.BlockSpec((BB, N), row),
        pl.BlockSpec((BB, N), row),
        pl.BlockSpec((D, D), full2),
        pl.BlockSpec((D,), lambda i: (0,)),
        pl.BlockSpec((D, D), full2),
        pl.BlockSpec((D,), lambda i: (0,)),
        pl.BlockSpec((4, D), full2),
        pl.BlockSpec((4, 1), full2),
        pl.BlockSpec((2, D), full2),
        pl.BlockSpec((2, 1), full2),
    ]
    out_specs = [
        pl.BlockSpec((BB, N, D), big),
        pl.BlockSpec((BB, N, D), big),
        pl.BlockSpec((BB, N), row),
        pl.BlockSpec((BB, N), row),
    ] + [pl.BlockSpec((BB, 1), row)] * 6
    return pl.pallas_call(
        _sbmem_kernel,
        grid=grid,
        in_specs=in_specs,
        out_specs=out_specs,
        out_shape=out_shapes,
        compiler_params=pltpu.CompilerParams(
            dimension_semantics=("arbitrary",)),
        interpret=interpret,
    )(hidden, working_keys, working_values, working_protection,
      working_usage, working_age, Wk, bk, Wv, bv, Wg, bg, Wsx, bsx)


def kernel(hidden, working_keys, working_values, working_protection,
           working_usage, working_age, Wk, bk, Wv, bv, Wwg, bwg, Wmg, bmg,
           Wbg, bbg, Wig, big, Wso, bso, Wsp, bsp):
    Wg = jnp.concatenate([Wwg, Wmg, Wbg, Wig], axis=0)
    bg = jnp.stack([bwg, bmg, bbg, big], axis=0)
    Wsx = jnp.concatenate([Wso, Wsp], axis=0)
    bsx = jnp.stack([bso, bsp], axis=0)
    (uk, uv, uprot, ow, ws, mp, bs, owr, pm, ms) = _run(
        hidden, working_keys, working_values, working_protection,
        working_usage, working_age, Wk, bk, Wv, bv, Wg, bg, Wsx, bsx)
    stats = {
        'write_strength': ws[:, 0],
        'merge_preference': mp[:, 0],
        'binding_strength': bs[:, 0],
        'overwrite_ratio': owr[:, 0],
        'protection_mean': pm[:, 0],
        'max_similarity': ms[:, 0],
        'slot_write_mass': ow,
    }
    return (uk, uv, uprot, stats)


# speedup vs baseline: 1.3014x; 1.3014x over previous
import functools

import jax
import jax.numpy as jnp
from jax.experimental import pallas as pl
from jax.experimental.pallas import tpu as pltpu

B, N, D = 1024, 256, 128
PROT_DECAY = 0.95
TEMP = 0.5
BB = 16
SB = 8

_bf16 = jnp.bfloat16
_f32 = jnp.float32


def _dot_t(a, b):
    return jax.lax.dot_general(
        a.astype(_bf16), b.astype(_bf16), (((1,), (1,)), ((), ())),
        preferred_element_type=_f32)


def _softmax_argmax(x, iota):
    m = jnp.max(x, axis=-1, keepdims=True)
    e = jnp.exp(x - m)
    s = jnp.sum(e, axis=-1, keepdims=True)
    d = e / s
    dmax = 1.0 / s
    return jnp.min(jnp.where(d == dmax, iota, N), axis=-1, keepdims=True)


def _sbmem_kernel(h_ref, wk_ref, wv_ref, prot_ref, usage_ref, age_ref,
                  Wk_ref, bk_ref, Wv_ref, bv_ref,
                  Wg_ref, bg_ref, Wsx_ref, bsx_ref,
                  uk_ref, uv_ref, uprot_ref, ow_ref,
                  ws_ref, mp_ref, bs_ref, owr_ref, pm_ref, ms_ref):
    h = h_ref[...]
    ck = jnp.tanh(_dot_t(h, Wk_ref[...]) + bk_ref[...][None, :])
    cv = jnp.tanh(_dot_t(h, Wv_ref[...]) + bv_ref[...][None, :])

    gates = _dot_t(h, Wg_ref[...]) + bg_ref[...][:, 0][None, :]
    write_strength = jax.nn.sigmoid(gates[:, 0:1])
    g_mg = gates[:, 1:2]
    g_bg = gates[:, 2:3]
    importance = jax.nn.sigmoid(gates[:, 3:4])

    ck_norm = jnp.sqrt(jnp.sum(ck * ck, axis=-1, keepdims=True))
    ncand = ck / jnp.maximum(ck_norm, 1e-6)

    wk = wk_ref[...]
    key_norm = jnp.sqrt(jnp.sum(wk * wk, axis=-1, keepdims=True))
    nkeys = wk / jnp.maximum(key_norm, 1e-6)
    nkeys_bf = nkeys.astype(_bf16)
    ncand_bf = ncand.astype(_bf16)
    bsel = jax.lax.broadcasted_iota(jnp.int32, (SB, 1, SB), 2) == \
        jax.lax.broadcasted_iota(jnp.int32, (SB, 1, SB), 0)
    sim_chunks = []
    for c in range(BB // SB):
        sa = jax.lax.dot_general(
            nkeys_bf[c * SB:(c + 1) * SB].reshape(SB * N, D),
            ncand_bf[c * SB:(c + 1) * SB], (((1,), (1,)), ((), ())),
            preferred_element_type=_f32).reshape(SB, N, SB)
        sim_chunks.append(jnp.sum(jnp.where(bsel, sa, 0.0), axis=-1))
    sim = jnp.concatenate(sim_chunks, axis=0)

    wv = wv_ref[...]
    val_norm = jnp.sqrt(jnp.sum(wv * wv, axis=-1))
    norm_occ = jnp.clip(val_norm / (D ** 0.5), 0.0, 1.0)
    sx = jax.lax.dot_general(
        wv.reshape(BB * N, D).astype(_bf16), Wsx_ref[...].astype(_bf16),
        (((1,), (1,)), ((), ())), preferred_element_type=_f32)
    learned_occ = jax.nn.sigmoid(sx[:, 0] + bsx_ref[0, 0]).reshape(BB, N)
    learned_prot = jax.nn.sigmoid(sx[:, 1] + bsx_ref[1, 0]).reshape(BB, N)
    occupancy = jnp.clip(0.5 * learned_occ + 0.5 * norm_occ, 0.0, 1.0)
    eff_prot = jnp.clip(0.4 * learned_prot + 0.6 * prot_ref[...], 0.0, 1.0)
    eff_usage = jnp.clip(0.5 * occupancy + 0.5 * usage_ref[...], 0.0, 1.0)
    eff_age = jnp.clip(age_ref[...], 0.0, 1.0)

    replace_scores = (1.15 * (1.0 - occupancy) + 0.85 * (1.0 - eff_prot)
                      + 0.65 * eff_age + 0.45 * (1.0 - eff_usage)
                      + 0.25 * (1.0 - sim))

    iota = jax.lax.broadcasted_iota(jnp.int32, (BB, N), 1)
    merge_index = _softmax_argmax(sim / TEMP, iota)
    replace_index = _softmax_argmax(replace_scores / TEMP, iota)

    merge_oh = (iota == merge_index).astype(_f32)
    max_sim = jnp.sum(sim * merge_oh, axis=-1, keepdims=True)
    matched_occ = jnp.sum(occupancy * merge_oh, axis=-1, keepdims=True)
    matched_usage = jnp.sum(eff_usage * merge_oh, axis=-1, keepdims=True)
    matched_age = jnp.sum(eff_age * merge_oh, axis=-1, keepdims=True)

    merge_pref = jax.nn.sigmoid(g_mg + 2.4 * max_sim
                                + 1.6 * (matched_occ - 0.5)
                                + 1.0 * (matched_usage - 0.5)
                                - 0.8 * matched_age)
    merge_cand = (max_sim > 0.55) & (matched_occ > 0.35)
    use_merge = (merge_pref >= 0.5) & merge_cand
    target_index = jnp.where(use_merge, merge_index, replace_index)
    target_oh = (iota == target_index).astype(_f32)

    binding = jax.nn.sigmoid(g_bg + 2.2 * max_sim)
    conflict = jnp.clip(1.0 - sim, 0.0, 1.0)
    overwrite = ((0.15 + 0.85 * write_strength) * target_oh
                 * (1.0 - 0.65 * eff_prot * conflict))

    key_mix = jnp.where(use_merge, 0.22 + 0.38 * binding, 0.78 + 0.18 * binding)
    value_mix = jnp.where(use_merge, 0.45 + 0.35 * importance, 0.75 + 0.2 * importance)

    owk = (overwrite * key_mix)[:, :, None]
    owv = (overwrite * value_mix)[:, :, None]
    uk_ref[...] = wk + owk * (ck[:, None, :] - wk)
    uv_ref[...] = wv + owv * (cv[:, None, :] - wv)

    prot_boost = overwrite * (0.5 + 0.5 * importance)
    uprot = jnp.clip(prot_ref[...] * PROT_DECAY + prot_boost, 0.0, 1.0)
    uprot_ref[...] = uprot

    ow_ref[...] = overwrite
    ws_ref[...] = write_strength
    mp_ref[...] = merge_pref
    bs_ref[...] = binding
    owr_ref[...] = jnp.mean(overwrite, axis=-1, keepdims=True)
    pm_ref[...] = jnp.mean(uprot, axis=-1, keepdims=True)
    ms_ref[...] = max_sim


@functools.partial(jax.jit, static_argnames=("interpret",))
def _run(hidden, working_keys, working_values, working_protection,
         working_usage, working_age, Wk, bk, Wv, bv, Wg, bg, Wsx, bsx,
         interpret=False):
    grid = (B // BB,)
    big = lambda i: (i, 0, 0)
    row = lambda i: (i, 0)
    full2 = lambda i: (0, 0)
    out_shapes = [
        jax.ShapeDtypeStruct((B, N, D), jnp.float32),
        jax.ShapeDtypeStruct((B, N, D), jnp.float32),
        jax.ShapeDtypeStruct((B, N), jnp.float32),
        jax.ShapeDtypeStruct((B, N), jnp.float32),
    ] + [jax.ShapeDtypeStruct((B, 1), jnp.float32)] * 6
    in_specs = [
        pl.BlockSpec((BB, D), row),
        pl.BlockSpec((BB, N, D), big),
        pl.BlockSpec((BB, N, D), big),
        pl.BlockSpec((BB, N), row),
        pl.BlockSpec((BB, N), row),
        pl.BlockSpec((BB, N), row),
        pl.BlockSpec((D, D), full2),
        pl.BlockSpec((D,), lambda i: (0,)),
        pl.BlockSpec((D, D), full2),
        pl.BlockSpec((D,), lambda i: (0,)),
        pl.BlockSpec((4, D), full2),
        pl.BlockSpec((4, 1), full2),
        pl.BlockSpec((2, D), full2),
        pl.BlockSpec((2, 1), full2),
    ]
    out_specs = [
        pl.BlockSpec((BB, N, D), big),
        pl.BlockSpec((BB, N, D), big),
        pl.BlockSpec((BB, N), row),
        pl.BlockSpec((BB, N), row),
    ] + [pl.BlockSpec((BB, 1), row)] * 6
    return pl.pallas_call(
        _sbmem_kernel,
        grid=grid,
        in_specs=in_specs,
        out_specs=out_specs,
        out_shape=out_shapes,
        compiler_params=pltpu.CompilerParams(
            dimension_semantics=("arbitrary",)),
        interpret=interpret,
    )(hidden, working_keys, working_values, working_protection,
      working_usage, working_age, Wk, bk, Wv, bv, Wg, bg, Wsx, bsx)


def kernel(hidden, working_keys, working_values, working_protection,
           working_usage, working_age, Wk, bk, Wv, bv, Wwg, bwg, Wmg, bmg,
           Wbg, bbg, Wig, big, Wso, bso, Wsp, bsp):
    Wg = jnp.concatenate([Wwg, Wmg, Wbg, Wig], axis=0)
    bg = jnp.stack([bwg, bmg, bbg, big], axis=0)
    Wsx = jnp.concatenate([Wso, Wsp], axis=0)
    bsx = jnp.stack([bso, bsp], axis=0)
    (uk, uv, uprot, ow, ws, mp, bs, owr, pm, ms) = _run(
        hidden, working_keys, working_values, working_protection,
        working_usage, working_age, Wk, bk, Wv, bv, Wg, bg, Wsx, bsx)
    stats = {
        'write_strength': ws[:, 0],
        'merge_preference': mp[:, 0],
        'binding_strength': bs[:, 0],
        'overwrite_ratio': owr[:, 0],
        'protection_mean': pm[:, 0],
        'max_similarity': ms[:, 0],
        'slot_write_mass': ow,
    }
    return (uk, uv, uprot, stats)


# speedup vs baseline: 3.8878x; 2.9874x over previous
import functools

import jax
import jax.numpy as jnp
from jax.experimental import pallas as pl
from jax.experimental.pallas import tpu as pltpu

B, N, D = 1024, 256, 128
PROT_DECAY = 0.95
TEMP = 0.5
BB = 16
SB = 8

_bf16 = jnp.bfloat16
_f32 = jnp.float32


def _dot_t(a, b):
    return jax.lax.dot_general(
        a.astype(_bf16), b.astype(_bf16), (((1,), (1,)), ((), ())),
        preferred_element_type=_f32)


def _softmax_argmax(x, iota):
    m = jnp.max(x, axis=-1, keepdims=True)
    e = jnp.exp(x - m)
    s = jnp.sum(e, axis=-1, keepdims=True)
    d = e / s
    dmax = 1.0 / s
    return jnp.min(jnp.where(d == dmax, iota, N), axis=-1, keepdims=True)


def _sbmem_kernel(h_ref, wk_ref, wv_ref, prot_ref, usage_ref, age_ref,
                  Wk_ref, bk_ref, Wv_ref, bv_ref,
                  Wg_ref, bg_ref, Wsx_ref, bsx_ref,
                  uk_ref, uv_ref, uprot_ref, ow_ref,
                  ws_ref, mp_ref, bs_ref, owr_ref, pm_ref, ms_ref):
    h = h_ref[...]
    ck = jnp.tanh(_dot_t(h, Wk_ref[...]) + bk_ref[...][None, :])
    cv = jnp.tanh(_dot_t(h, Wv_ref[...]) + bv_ref[...][None, :])

    gates = _dot_t(h, Wg_ref[...]) + bg_ref[...][:, 0][None, :]
    write_strength = jax.nn.sigmoid(gates[:, 0:1])
    g_mg = gates[:, 1:2]
    g_bg = gates[:, 2:3]
    importance = jax.nn.sigmoid(gates[:, 3:4])

    ck_norm = jnp.sqrt(jnp.sum(ck * ck, axis=-1, keepdims=True))
    ncand = ck / jnp.maximum(ck_norm, 1e-6)

    wk = wk_ref[...]
    key_norm = jnp.sqrt(jnp.sum(wk * wk, axis=-1, keepdims=True))
    nkeys = wk / jnp.maximum(key_norm, 1e-6)
    sim_all = jax.lax.dot_general(
        ncand.astype(_bf16), nkeys.astype(_bf16).reshape(BB * N, D),
        (((1,), (1,)), ((), ())),
        preferred_element_type=_f32).reshape(BB, BB, N)
    bsel = (jax.lax.broadcasted_iota(jnp.int32, (BB, BB, 1), 0) ==
            jax.lax.broadcasted_iota(jnp.int32, (BB, BB, 1), 1))
    sim = jnp.sum(jnp.where(bsel, sim_all, 0.0), axis=1)

    wv = wv_ref[...]
    val_norm = jnp.sqrt(jnp.sum(wv * wv, axis=-1))
    norm_occ = jnp.clip(val_norm / (D ** 0.5), 0.0, 1.0)
    sx = jax.lax.dot_general(
        Wsx_ref[...].astype(_bf16), wv.astype(_bf16).reshape(BB * N, D),
        (((1,), (1,)), ((), ())),
        preferred_element_type=_f32).reshape(2, BB, N)
    learned_occ = jax.nn.sigmoid(sx[0] + bsx_ref[0, 0])
    learned_prot = jax.nn.sigmoid(sx[1] + bsx_ref[1, 0])
    occupancy = jnp.clip(0.5 * learned_occ + 0.5 * norm_occ, 0.0, 1.0)
    eff_prot = jnp.clip(0.4 * learned_prot + 0.6 * prot_ref[...], 0.0, 1.0)
    eff_usage = jnp.clip(0.5 * occupancy + 0.5 * usage_ref[...], 0.0, 1.0)
    eff_age = jnp.clip(age_ref[...], 0.0, 1.0)

    replace_scores = (1.15 * (1.0 - occupancy) + 0.85 * (1.0 - eff_prot)
                      + 0.65 * eff_age + 0.45 * (1.0 - eff_usage)
                      + 0.25 * (1.0 - sim))

    iota = jax.lax.broadcasted_iota(jnp.int32, (BB, N), 1)
    merge_index = _softmax_argmax(sim / TEMP, iota)
    replace_index = _softmax_argmax(replace_scores / TEMP, iota)

    merge_oh = (iota == merge_index).astype(_f32)
    max_sim = jnp.sum(sim * merge_oh, axis=-1, keepdims=True)
    matched_occ = jnp.sum(occupancy * merge_oh, axis=-1, keepdims=True)
    matched_usage = jnp.sum(eff_usage * merge_oh, axis=-1, keepdims=True)
    matched_age = jnp.sum(eff_age * merge_oh, axis=-1, keepdims=True)

    merge_pref = jax.nn.sigmoid(g_mg + 2.4 * max_sim
                                + 1.6 * (matched_occ - 0.5)
                                + 1.0 * (matched_usage - 0.5)
                                - 0.8 * matched_age)
    merge_cand = (max_sim > 0.55) & (matched_occ > 0.35)
    use_merge = (merge_pref >= 0.5) & merge_cand
    target_index = jnp.where(use_merge, merge_index, replace_index)
    target_oh = (iota == target_index).astype(_f32)

    binding = jax.nn.sigmoid(g_bg + 2.2 * max_sim)
    conflict = jnp.clip(1.0 - sim, 0.0, 1.0)
    overwrite = ((0.15 + 0.85 * write_strength) * target_oh
                 * (1.0 - 0.65 * eff_prot * conflict))

    key_mix = jnp.where(use_merge, 0.22 + 0.38 * binding, 0.78 + 0.18 * binding)
    value_mix = jnp.where(use_merge, 0.45 + 0.35 * importance, 0.75 + 0.2 * importance)

    owk = (overwrite * key_mix)[:, :, None]
    owv = (overwrite * value_mix)[:, :, None]
    uk_ref[...] = wk + owk * (ck[:, None, :] - wk)
    uv_ref[...] = wv + owv * (cv[:, None, :] - wv)

    prot_boost = overwrite * (0.5 + 0.5 * importance)
    uprot = jnp.clip(prot_ref[...] * PROT_DECAY + prot_boost, 0.0, 1.0)
    uprot_ref[...] = uprot

    ow_ref[...] = overwrite
    ws_ref[...] = write_strength
    mp_ref[...] = merge_pref
    bs_ref[...] = binding
    owr_ref[...] = jnp.mean(overwrite, axis=-1, keepdims=True)
    pm_ref[...] = jnp.mean(uprot, axis=-1, keepdims=True)
    ms_ref[...] = max_sim


@functools.partial(jax.jit, static_argnames=("interpret",))
def _run(hidden, working_keys, working_values, working_protection,
         working_usage, working_age, Wk, bk, Wv, bv, Wg, bg, Wsx, bsx,
         interpret=False):
    grid = (B // BB,)
    big = lambda i: (i, 0, 0)
    row = lambda i: (i, 0)
    full2 = lambda i: (0, 0)
    out_shapes = [
        jax.ShapeDtypeStruct((B, N, D), jnp.float32),
        jax.ShapeDtypeStruct((B, N, D), jnp.float32),
        jax.ShapeDtypeStruct((B, N), jnp.float32),
        jax.ShapeDtypeStruct((B, N), jnp.float32),
    ] + [jax.ShapeDtypeStruct((B, 1), jnp.float32)] * 6
    in_specs = [
        pl.BlockSpec((BB, D), row),
        pl.BlockSpec((BB, N, D), big),
        pl.BlockSpec((BB, N, D), big),
        pl.BlockSpec((BB, N), row),
        pl.BlockSpec((BB, N), row),
        pl.BlockSpec((BB, N), row),
        pl.BlockSpec((D, D), full2),
        pl.BlockSpec((D,), lambda i: (0,)),
        pl.BlockSpec((D, D), full2),
        pl.BlockSpec((D,), lambda i: (0,)),
        pl.BlockSpec((4, D), full2),
        pl.BlockSpec((4, 1), full2),
        pl.BlockSpec((2, D), full2),
        pl.BlockSpec((2, 1), full2),
    ]
    out_specs = [
        pl.BlockSpec((BB, N, D), big),
        pl.BlockSpec((BB, N, D), big),
        pl.BlockSpec((BB, N), row),
        pl.BlockSpec((BB, N), row),
    ] + [pl.BlockSpec((BB, 1), row)] * 6
    return pl.pallas_call(
        _sbmem_kernel,
        grid=grid,
        in_specs=in_specs,
        out_specs=out_specs,
        out_shape=out_shapes,
        compiler_params=pltpu.CompilerParams(
            dimension_semantics=("arbitrary",)),
        interpret=interpret,
    )(hidden, working_keys, working_values, working_protection,
      working_usage, working_age, Wk, bk, Wv, bv, Wg, bg, Wsx, bsx)


def kernel(hidden, working_keys, working_values, working_protection,
           working_usage, working_age, Wk, bk, Wv, bv, Wwg, bwg, Wmg, bmg,
           Wbg, bbg, Wig, big, Wso, bso, Wsp, bsp):
    Wg = jnp.concatenate([Wwg, Wmg, Wbg, Wig], axis=0)
    bg = jnp.stack([bwg, bmg, bbg, big], axis=0)
    Wsx = jnp.concatenate([Wso, Wsp], axis=0)
    bsx = jnp.stack([bso, bsp], axis=0)
    (uk, uv, uprot, ow, ws, mp, bs, owr, pm, ms) = _run(
        hidden, working_keys, working_values, working_protection,
        working_usage, working_age, Wk, bk, Wv, bv, Wg, bg, Wsx, bsx)
    stats = {
        'write_strength': ws[:, 0],
        'merge_preference': mp[:, 0],
        'binding_strength': bs[:, 0],
        'overwrite_ratio': owr[:, 0],
        'protection_mean': pm[:, 0],
        'max_similarity': ms[:, 0],
        'slot_write_mass': ow,
    }
    return (uk, uv, uprot, stats)


# speedup vs baseline: 4.2518x; 1.0936x over previous
import functools

import jax
import jax.numpy as jnp
from jax.experimental import pallas as pl
from jax.experimental.pallas import tpu as pltpu

B, N, D = 1024, 256, 128
PROT_DECAY = 0.95
TEMP = 0.5
BB = 32
SB = 8

_bf16 = jnp.bfloat16
_f32 = jnp.float32


def _dot_t(a, b):
    return jax.lax.dot_general(
        a.astype(_bf16), b.astype(_bf16), (((1,), (1,)), ((), ())),
        preferred_element_type=_f32)


def _softmax_argmax(x, iota):
    m = jnp.max(x, axis=-1, keepdims=True)
    e = jnp.exp(x - m)
    s = jnp.sum(e, axis=-1, keepdims=True)
    d = e / s
    dmax = 1.0 / s
    return jnp.min(jnp.where(d == dmax, iota, N), axis=-1, keepdims=True)


def _sbmem_kernel(h_ref, wk_ref, wv_ref, prot_ref, usage_ref, age_ref,
                  Wk_ref, bk_ref, Wv_ref, bv_ref,
                  Wg_ref, bg_ref, Wsx_ref, bsx_ref,
                  uk_ref, uv_ref, uprot_ref, ow_ref,
                  ws_ref, mp_ref, bs_ref, owr_ref, pm_ref, ms_ref):
    h = h_ref[...]
    ck = jnp.tanh(_dot_t(h, Wk_ref[...]) + bk_ref[...][None, :])
    cv = jnp.tanh(_dot_t(h, Wv_ref[...]) + bv_ref[...][None, :])

    gates = _dot_t(h, Wg_ref[...]) + bg_ref[...][:, 0][None, :]
    write_strength = jax.nn.sigmoid(gates[:, 0:1])
    g_mg = gates[:, 1:2]
    g_bg = gates[:, 2:3]
    importance = jax.nn.sigmoid(gates[:, 3:4])

    ck_norm = jnp.sqrt(jnp.sum(ck * ck, axis=-1, keepdims=True))
    ncand = ck / jnp.maximum(ck_norm, 1e-6)

    wk = wk_ref[...]
    key_norm = jnp.sqrt(jnp.sum(wk * wk, axis=-1, keepdims=True))
    nkeys = wk / jnp.maximum(key_norm, 1e-6)
    sim_all = jax.lax.dot_general(
        ncand.astype(_bf16), nkeys.astype(_bf16).reshape(BB * N, D),
        (((1,), (1,)), ((), ())),
        preferred_element_type=_f32).reshape(BB, BB, N)
    bsel = (jax.lax.broadcasted_iota(jnp.int32, (BB, BB, 1), 0) ==
            jax.lax.broadcasted_iota(jnp.int32, (BB, BB, 1), 1))
    sim = jnp.sum(jnp.where(bsel, sim_all, 0.0), axis=1)

    wv = wv_ref[...]
    val_norm = jnp.sqrt(jnp.sum(wv * wv, axis=-1))
    norm_occ = jnp.clip(val_norm / (D ** 0.5), 0.0, 1.0)
    sx = jax.lax.dot_general(
        Wsx_ref[...].astype(_bf16), wv.astype(_bf16).reshape(BB * N, D),
        (((1,), (1,)), ((), ())),
        preferred_element_type=_f32).reshape(2, BB, N)
    learned_occ = jax.nn.sigmoid(sx[0] + bsx_ref[0, 0])
    learned_prot = jax.nn.sigmoid(sx[1] + bsx_ref[1, 0])
    occupancy = jnp.clip(0.5 * learned_occ + 0.5 * norm_occ, 0.0, 1.0)
    eff_prot = jnp.clip(0.4 * learned_prot + 0.6 * prot_ref[...], 0.0, 1.0)
    eff_usage = jnp.clip(0.5 * occupancy + 0.5 * usage_ref[...], 0.0, 1.0)
    eff_age = jnp.clip(age_ref[...], 0.0, 1.0)

    replace_scores = (1.15 * (1.0 - occupancy) + 0.85 * (1.0 - eff_prot)
                      + 0.65 * eff_age + 0.45 * (1.0 - eff_usage)
                      + 0.25 * (1.0 - sim))

    iota = jax.lax.broadcasted_iota(jnp.int32, (BB, N), 1)
    merge_index = _softmax_argmax(sim / TEMP, iota)
    replace_index = _softmax_argmax(replace_scores / TEMP, iota)

    merge_oh = (iota == merge_index).astype(_f32)
    max_sim = jnp.sum(sim * merge_oh, axis=-1, keepdims=True)
    matched_occ = jnp.sum(occupancy * merge_oh, axis=-1, keepdims=True)
    matched_usage = jnp.sum(eff_usage * merge_oh, axis=-1, keepdims=True)
    matched_age = jnp.sum(eff_age * merge_oh, axis=-1, keepdims=True)

    merge_pref = jax.nn.sigmoid(g_mg + 2.4 * max_sim
                                + 1.6 * (matched_occ - 0.5)
                                + 1.0 * (matched_usage - 0.5)
                                - 0.8 * matched_age)
    merge_cand = (max_sim > 0.55) & (matched_occ > 0.35)
    use_merge = (merge_pref >= 0.5) & merge_cand
    target_index = jnp.where(use_merge, merge_index, replace_index)
    target_oh = (iota == target_index).astype(_f32)

    binding = jax.nn.sigmoid(g_bg + 2.2 * max_sim)
    conflict = jnp.clip(1.0 - sim, 0.0, 1.0)
    overwrite = ((0.15 + 0.85 * write_strength) * target_oh
                 * (1.0 - 0.65 * eff_prot * conflict))

    key_mix = jnp.where(use_merge, 0.22 + 0.38 * binding, 0.78 + 0.18 * binding)
    value_mix = jnp.where(use_merge, 0.45 + 0.35 * importance, 0.75 + 0.2 * importance)

    owk = (overwrite * key_mix)[:, :, None]
    owv = (overwrite * value_mix)[:, :, None]
    uk_ref[...] = wk + owk * (ck[:, None, :] - wk)
    uv_ref[...] = wv + owv * (cv[:, None, :] - wv)

    prot_boost = overwrite * (0.5 + 0.5 * importance)
    uprot = jnp.clip(prot_ref[...] * PROT_DECAY + prot_boost, 0.0, 1.0)
    uprot_ref[...] = uprot

    ow_ref[...] = overwrite
    ws_ref[...] = write_strength
    mp_ref[...] = merge_pref
    bs_ref[...] = binding
    owr_ref[...] = jnp.mean(overwrite, axis=-1, keepdims=True)
    pm_ref[...] = jnp.mean(uprot, axis=-1, keepdims=True)
    ms_ref[...] = max_sim


@functools.partial(jax.jit, static_argnames=("interpret",))
def _run(hidden, working_keys, working_values, working_protection,
         working_usage, working_age, Wk, bk, Wv, bv, Wg, bg, Wsx, bsx,
         interpret=False):
    grid = (B // BB,)
    big = lambda i: (i, 0, 0)
    row = lambda i: (i, 0)
    full2 = lambda i: (0, 0)
    out_shapes = [
        jax.ShapeDtypeStruct((B, N, D), jnp.float32),
        jax.ShapeDtypeStruct((B, N, D), jnp.float32),
        jax.ShapeDtypeStruct((B, N), jnp.float32),
        jax.ShapeDtypeStruct((B, N), jnp.float32),
    ] + [jax.ShapeDtypeStruct((B, 1), jnp.float32)] * 6
    in_specs = [
        pl.BlockSpec((BB, D), row),
        pl.BlockSpec((BB, N, D), big),
        pl.BlockSpec((BB, N, D), big),
        pl.BlockSpec((BB, N), row),
        pl.BlockSpec((BB, N), row),
        pl.BlockSpec((BB, N), row),
        pl.BlockSpec((D, D), full2),
        pl.BlockSpec((D,), lambda i: (0,)),
        pl.BlockSpec((D, D), full2),
        pl.BlockSpec((D,), lambda i: (0,)),
        pl.BlockSpec((4, D), full2),
        pl.BlockSpec((4, 1), full2),
        pl.BlockSpec((2, D), full2),
        pl.BlockSpec((2, 1), full2),
    ]
    out_specs = [
        pl.BlockSpec((BB, N, D), big),
        pl.BlockSpec((BB, N, D), big),
        pl.BlockSpec((BB, N), row),
        pl.BlockSpec((BB, N), row),
    ] + [pl.BlockSpec((BB, 1), row)] * 6
    return pl.pallas_call(
        _sbmem_kernel,
        grid=grid,
        in_specs=in_specs,
        out_specs=out_specs,
        out_shape=out_shapes,
        compiler_params=pltpu.CompilerParams(
            dimension_semantics=("arbitrary",)),
        interpret=interpret,
    )(hidden, working_keys, working_values, working_protection,
      working_usage, working_age, Wk, bk, Wv, bv, Wg, bg, Wsx, bsx)


def kernel(hidden, working_keys, working_values, working_protection,
           working_usage, working_age, Wk, bk, Wv, bv, Wwg, bwg, Wmg, bmg,
           Wbg, bbg, Wig, big, Wso, bso, Wsp, bsp):
    Wg = jnp.concatenate([Wwg, Wmg, Wbg, Wig], axis=0)
    bg = jnp.stack([bwg, bmg, bbg, big], axis=0)
    Wsx = jnp.concatenate([Wso, Wsp], axis=0)
    bsx = jnp.stack([bso, bsp], axis=0)
    (uk, uv, uprot, ow, ws, mp, bs, owr, pm, ms) = _run(
        hidden, working_keys, working_values, working_protection,
        working_usage, working_age, Wk, bk, Wv, bv, Wg, bg, Wsx, bsx)
    stats = {
        'write_strength': ws[:, 0],
        'merge_preference': mp[:, 0],
        'binding_strength': bs[:, 0],
        'overwrite_ratio': owr[:, 0],
        'protection_mean': pm[:, 0],
        'max_similarity': ms[:, 0],
        'slot_write_mass': ow,
    }
    return (uk, uv, uprot, stats)


# speedup vs baseline: 4.3574x; 1.0248x over previous
import functools

import jax
import jax.numpy as jnp
from jax.experimental import pallas as pl
from jax.experimental.pallas import tpu as pltpu

B, N, D = 1024, 256, 128
PROT_DECAY = 0.95
TEMP = 0.5
BB = 32
SB = 8

_bf16 = jnp.bfloat16
_f32 = jnp.float32


def _dot_t(a, b):
    return jax.lax.dot_general(
        a.astype(_bf16), b.astype(_bf16), (((1,), (1,)), ((), ())),
        preferred_element_type=_f32)


def _softmax_argmax(x, iota):
    m = jnp.max(x, axis=-1, keepdims=True)
    e = jnp.exp(x - m)
    s = jnp.sum(e, axis=-1, keepdims=True)
    d = e / s
    dmax = 1.0 / s
    return jnp.min(jnp.where(d == dmax, iota, N), axis=-1, keepdims=True)


def _sbmem_kernel(h_ref, wk_ref, wv_ref, prot_ref, usage_ref, age_ref,
                  Wk_ref, bk_ref, Wv_ref, bv_ref,
                  Wg_ref, bg_ref, Wsx_ref, bsx_ref,
                  uk_ref, uv_ref, uprot_ref, ow_ref,
                  ws_ref, mp_ref, bs_ref, owr_ref, pm_ref, ms_ref):
    h = h_ref[...]
    ck = jnp.tanh(_dot_t(h, Wk_ref[...]) + bk_ref[...][None, :])
    cv = jnp.tanh(_dot_t(h, Wv_ref[...]) + bv_ref[...][None, :])

    gates = _dot_t(h, Wg_ref[...]) + bg_ref[...][:, 0][None, :]
    write_strength = jax.nn.sigmoid(gates[:, 0:1])
    g_mg = gates[:, 1:2]
    g_bg = gates[:, 2:3]
    importance = jax.nn.sigmoid(gates[:, 3:4])

    ck_norm = jnp.sqrt(jnp.sum(ck * ck, axis=-1, keepdims=True))
    ncand = ck / jnp.maximum(ck_norm, 1e-6)

    wk = wk_ref[...]
    key_norm = jnp.sqrt(jnp.sum(wk * wk, axis=-1, keepdims=True))
    nkeys = wk / jnp.maximum(key_norm, 1e-6)
    nkeys_bf = nkeys.astype(_bf16)
    ncand_bf = ncand.astype(_bf16)
    bsel = (jax.lax.broadcasted_iota(jnp.int32, (SB, SB, 1), 0) ==
            jax.lax.broadcasted_iota(jnp.int32, (SB, SB, 1), 1))
    sim_chunks = []
    for c in range(BB // SB):
        sa = jax.lax.dot_general(
            ncand_bf[c * SB:(c + 1) * SB],
            nkeys_bf[c * SB:(c + 1) * SB].reshape(SB * N, D),
            (((1,), (1,)), ((), ())),
            preferred_element_type=_f32).reshape(SB, SB, N)
        sim_chunks.append(jnp.sum(jnp.where(bsel, sa, 0.0), axis=1))
    sim = jnp.concatenate(sim_chunks, axis=0)

    wv = wv_ref[...]
    val_norm = jnp.sqrt(jnp.sum(wv * wv, axis=-1))
    norm_occ = jnp.clip(val_norm / (D ** 0.5), 0.0, 1.0)
    sx = jax.lax.dot_general(
        Wsx_ref[...].astype(_bf16), wv.astype(_bf16).reshape(BB * N, D),
        (((1,), (1,)), ((), ())),
        preferred_element_type=_f32).reshape(2, BB, N)
    learned_occ = jax.nn.sigmoid(sx[0] + bsx_ref[0, 0])
    learned_prot = jax.nn.sigmoid(sx[1] + bsx_ref[1, 0])
    occupancy = jnp.clip(0.5 * learned_occ + 0.5 * norm_occ, 0.0, 1.0)
    eff_prot = jnp.clip(0.4 * learned_prot + 0.6 * prot_ref[...], 0.0, 1.0)
    eff_usage = jnp.clip(0.5 * occupancy + 0.5 * usage_ref[...], 0.0, 1.0)
    eff_age = jnp.clip(age_ref[...], 0.0, 1.0)

    replace_scores = (1.15 * (1.0 - occupancy) + 0.85 * (1.0 - eff_prot)
                      + 0.65 * eff_age + 0.45 * (1.0 - eff_usage)
                      + 0.25 * (1.0 - sim))

    iota = jax.lax.broadcasted_iota(jnp.int32, (BB, N), 1)
    merge_index = _softmax_argmax(sim / TEMP, iota)
    replace_index = _softmax_argmax(replace_scores / TEMP, iota)

    merge_oh = (iota == merge_index).astype(_f32)
    max_sim = jnp.sum(sim * merge_oh, axis=-1, keepdims=True)
    matched_occ = jnp.sum(occupancy * merge_oh, axis=-1, keepdims=True)
    matched_usage = jnp.sum(eff_usage * merge_oh, axis=-1, keepdims=True)
    matched_age = jnp.sum(eff_age * merge_oh, axis=-1, keepdims=True)

    merge_pref = jax.nn.sigmoid(g_mg + 2.4 * max_sim
                                + 1.6 * (matched_occ - 0.5)
                                + 1.0 * (matched_usage - 0.5)
                                - 0.8 * matched_age)
    merge_cand = (max_sim > 0.55) & (matched_occ > 0.35)
    use_merge = (merge_pref >= 0.5) & merge_cand
    target_index = jnp.where(use_merge, merge_index, replace_index)
    target_oh = (iota == target_index).astype(_f32)

    binding = jax.nn.sigmoid(g_bg + 2.2 * max_sim)
    conflict = jnp.clip(1.0 - sim, 0.0, 1.0)
    overwrite = ((0.15 + 0.85 * write_strength) * target_oh
                 * (1.0 - 0.65 * eff_prot * conflict))

    key_mix = jnp.where(use_merge, 0.22 + 0.38 * binding, 0.78 + 0.18 * binding)
    value_mix = jnp.where(use_merge, 0.45 + 0.35 * importance, 0.75 + 0.2 * importance)

    owk = (overwrite * key_mix)[:, :, None]
    owv = (overwrite * value_mix)[:, :, None]
    uk_ref[...] = wk + owk * (ck[:, None, :] - wk)
    uv_ref[...] = wv + owv * (cv[:, None, :] - wv)

    prot_boost = overwrite * (0.5 + 0.5 * importance)
    uprot = jnp.clip(prot_ref[...] * PROT_DECAY + prot_boost, 0.0, 1.0)
    uprot_ref[...] = uprot

    ow_ref[...] = overwrite
    ws_ref[...] = write_strength
    mp_ref[...] = merge_pref
    bs_ref[...] = binding
    owr_ref[...] = jnp.mean(overwrite, axis=-1, keepdims=True)
    pm_ref[...] = jnp.mean(uprot, axis=-1, keepdims=True)
    ms_ref[...] = max_sim


@functools.partial(jax.jit, static_argnames=("interpret",))
def _run(hidden, working_keys, working_values, working_protection,
         working_usage, working_age, Wk, bk, Wv, bv, Wg, bg, Wsx, bsx,
         interpret=False):
    grid = (B // BB,)
    big = lambda i: (i, 0, 0)
    row = lambda i: (i, 0)
    full2 = lambda i: (0, 0)
    out_shapes = [
        jax.ShapeDtypeStruct((B, N, D), jnp.float32),
        jax.ShapeDtypeStruct((B, N, D), jnp.float32),
        jax.ShapeDtypeStruct((B, N), jnp.float32),
        jax.ShapeDtypeStruct((B, N), jnp.float32),
    ] + [jax.ShapeDtypeStruct((B, 1), jnp.float32)] * 6
    in_specs = [
        pl.BlockSpec((BB, D), row),
        pl.BlockSpec((BB, N, D), big),
        pl.BlockSpec((BB, N, D), big),
        pl.BlockSpec((BB, N), row),
        pl.BlockSpec((BB, N), row),
        pl.BlockSpec((BB, N), row),
        pl.BlockSpec((D, D), full2),
        pl.BlockSpec((D,), lambda i: (0,)),
        pl.BlockSpec((D, D), full2),
        pl.BlockSpec((D,), lambda i: (0,)),
        pl.BlockSpec((4, D), full2),
        pl.BlockSpec((4, 1), full2),
        pl.BlockSpec((2, D), full2),
        pl.BlockSpec((2, 1), full2),
    ]
    out_specs = [
        pl.BlockSpec((BB, N, D), big),
        pl.BlockSpec((BB, N, D), big),
        pl.BlockSpec((BB, N), row),
        pl.BlockSpec((BB, N), row),
    ] + [pl.BlockSpec((BB, 1), row)] * 6
    return pl.pallas_call(
        _sbmem_kernel,
        grid=grid,
        in_specs=in_specs,
        out_specs=out_specs,
        out_shape=out_shapes,
        compiler_params=pltpu.CompilerParams(
            dimension_semantics=("arbitrary",)),
        interpret=interpret,
    )(hidden, working_keys, working_values, working_protection,
      working_usage, working_age, Wk, bk, Wv, bv, Wg, bg, Wsx, bsx)


def kernel(hidden, working_keys, working_values, working_protection,
           working_usage, working_age, Wk, bk, Wv, bv, Wwg, bwg, Wmg, bmg,
           Wbg, bbg, Wig, big, Wso, bso, Wsp, bsp):
    Wg = jnp.concatenate([Wwg, Wmg, Wbg, Wig], axis=0)
    bg = jnp.stack([bwg, bmg, bbg, big], axis=0)
    Wsx = jnp.concatenate([Wso, Wsp], axis=0)
    bsx = jnp.stack([bso, bsp], axis=0)
    (uk, uv, uprot, ow, ws, mp, bs, owr, pm, ms) = _run(
        hidden, working_keys, working_values, working_protection,
        working_usage, working_age, Wk, bk, Wv, bv, Wg, bg, Wsx, bsx)
    stats = {
        'write_strength': ws[:, 0],
        'merge_preference': mp[:, 0],
        'binding_strength': bs[:, 0],
        'overwrite_ratio': owr[:, 0],
        'protection_mean': pm[:, 0],
        'max_similarity': ms[:, 0],
        'slot_write_mass': ow,
    }
    return (uk, uv, uprot, stats)


# speedup vs baseline: 4.4008x; 1.0100x over previous
import functools

import jax
import jax.numpy as jnp
from jax.experimental import pallas as pl
from jax.experimental.pallas import tpu as pltpu

B, N, D = 1024, 256, 128
PROT_DECAY = 0.95
TEMP = 0.5
BB = 32
SB = 8

_bf16 = jnp.bfloat16
_f32 = jnp.float32


def _dot_t(a, b):
    return jax.lax.dot_general(
        a.astype(_bf16), b.astype(_bf16), (((1,), (1,)), ((), ())),
        preferred_element_type=_f32)


def _softmax_argmax(x, iota):
    m = jnp.max(x, axis=-1, keepdims=True)
    e = jnp.exp(x - m)
    s = jnp.sum(e, axis=-1, keepdims=True)
    d = e / s
    dmax = 1.0 / s
    return jnp.min(jnp.where(d == dmax, iota, N), axis=-1, keepdims=True)


def _sbmem_kernel(h_ref, wk_ref, wv_ref, pua_ref,
                  Wk_ref, bk_ref, Wv_ref, bv_ref,
                  Wg_ref, bg_ref, Wsx_ref, bsx_ref,
                  uk_ref, uv_ref, po_ref):
    prot_in = pua_ref[:, 0:N]
    usage_in = pua_ref[:, N:2 * N]
    age_in = pua_ref[:, 2 * N:3 * N]
    h = h_ref[...]
    ck = jnp.tanh(_dot_t(h, Wk_ref[...]) + bk_ref[...][None, :])
    cv = jnp.tanh(_dot_t(h, Wv_ref[...]) + bv_ref[...][None, :])

    gates = _dot_t(h, Wg_ref[...]) + bg_ref[...][:, 0][None, :]
    write_strength = jax.nn.sigmoid(gates[:, 0:1])
    g_mg = gates[:, 1:2]
    g_bg = gates[:, 2:3]
    importance = jax.nn.sigmoid(gates[:, 3:4])

    ck_norm = jnp.sqrt(jnp.sum(ck * ck, axis=-1, keepdims=True))
    ncand = ck / jnp.maximum(ck_norm, 1e-6)

    wk = wk_ref[...]
    key_norm = jnp.sqrt(jnp.sum(wk * wk, axis=-1))
    nkeys = wk / jnp.maximum(key_norm, 1e-6)[:, :, None]
    nkeys_bf = nkeys.astype(_bf16)
    ncand_bf = ncand.astype(_bf16)
    bsel = (jax.lax.broadcasted_iota(jnp.int32, (SB, SB, 1), 0) ==
            jax.lax.broadcasted_iota(jnp.int32, (SB, SB, 1), 1))
    sim_chunks = []
    for c in range(BB // SB):
        sa = jax.lax.dot_general(
            ncand_bf[c * SB:(c + 1) * SB],
            nkeys_bf[c * SB:(c + 1) * SB].reshape(SB * N, D),
            (((1,), (1,)), ((), ())),
            preferred_element_type=_f32).reshape(SB, SB, N)
        sim_chunks.append(jnp.sum(jnp.where(bsel, sa, 0.0), axis=1))
    sim = jnp.concatenate(sim_chunks, axis=0)

    wv = wv_ref[...]
    val_norm = jnp.sqrt(jnp.sum(wv * wv, axis=-1))
    norm_occ = jnp.clip(val_norm / (D ** 0.5), 0.0, 1.0)
    sx = jax.lax.dot_general(
        Wsx_ref[...].astype(_bf16), wv.astype(_bf16).reshape(BB * N, D),
        (((1,), (1,)), ((), ())),
        preferred_element_type=_f32).reshape(2, BB, N)
    learned_occ = jax.nn.sigmoid(sx[0] + bsx_ref[0, 0])
    learned_prot = jax.nn.sigmoid(sx[1] + bsx_ref[1, 0])
    occupancy = jnp.clip(0.5 * learned_occ + 0.5 * norm_occ, 0.0, 1.0)
    eff_prot = jnp.clip(0.4 * learned_prot + 0.6 * prot_in, 0.0, 1.0)
    eff_usage = jnp.clip(0.5 * occupancy + 0.5 * usage_in, 0.0, 1.0)
    eff_age = jnp.clip(age_in, 0.0, 1.0)

    replace_scores = (1.15 * (1.0 - occupancy) + 0.85 * (1.0 - eff_prot)
                      + 0.65 * eff_age + 0.45 * (1.0 - eff_usage)
                      + 0.25 * (1.0 - sim))

    iota = jax.lax.broadcasted_iota(jnp.int32, (BB, N), 1)
    merge_index = _softmax_argmax(sim / TEMP, iota)
    replace_index = _softmax_argmax(replace_scores / TEMP, iota)

    merge_oh = (iota == merge_index).astype(_f32)
    max_sim = jnp.sum(sim * merge_oh, axis=-1, keepdims=True)
    matched_occ = jnp.sum(occupancy * merge_oh, axis=-1, keepdims=True)
    matched_usage = jnp.sum(eff_usage * merge_oh, axis=-1, keepdims=True)
    matched_age = jnp.sum(eff_age * merge_oh, axis=-1, keepdims=True)

    merge_pref = jax.nn.sigmoid(g_mg + 2.4 * max_sim
                                + 1.6 * (matched_occ - 0.5)
                                + 1.0 * (matched_usage - 0.5)
                                - 0.8 * matched_age)
    merge_cand = (max_sim > 0.55) & (matched_occ > 0.35)
    use_merge = (merge_pref >= 0.5) & merge_cand
    target_index = jnp.where(use_merge, merge_index, replace_index)
    target_oh = (iota == target_index).astype(_f32)

    binding = jax.nn.sigmoid(g_bg + 2.2 * max_sim)
    conflict = jnp.clip(1.0 - sim, 0.0, 1.0)
    overwrite = ((0.15 + 0.85 * write_strength) * target_oh
                 * (1.0 - 0.65 * eff_prot * conflict))

    key_mix = jnp.where(use_merge, 0.22 + 0.38 * binding, 0.78 + 0.18 * binding)
    value_mix = jnp.where(use_merge, 0.45 + 0.35 * importance, 0.75 + 0.2 * importance)

    owk = (overwrite * key_mix)[:, :, None]
    owv = (overwrite * value_mix)[:, :, None]
    uk_ref[...] = wk + owk * (ck[:, None, :] - wk)
    uv_ref[...] = wv + owv * (cv[:, None, :] - wv)

    prot_boost = overwrite * (0.5 + 0.5 * importance)
    uprot = jnp.clip(prot_in * PROT_DECAY + prot_boost, 0.0, 1.0)
    stcat = jnp.concatenate(
        [write_strength, merge_pref, binding,
         jnp.mean(overwrite, axis=-1, keepdims=True),
         jnp.mean(uprot, axis=-1, keepdims=True), max_sim,
         jnp.zeros((BB, D - 6), _f32)], axis=1)
    po_ref[...] = jnp.concatenate([uprot, overwrite, stcat], axis=1)


@functools.partial(jax.jit, static_argnames=("interpret",))
def _run(hidden, working_keys, working_values, pua,
         Wk, bk, Wv, bv, Wg, bg, Wsx, bsx,
         interpret=False):
    grid = (B // BB,)
    big = lambda i: (i, 0, 0)
    row = lambda i: (i, 0)
    full2 = lambda i: (0, 0)
    out_shapes = [
        jax.ShapeDtypeStruct((B, N, D), jnp.float32),
        jax.ShapeDtypeStruct((B, N, D), jnp.float32),
        jax.ShapeDtypeStruct((B, 2 * N + D), jnp.float32),
    ]
    in_specs = [
        pl.BlockSpec((BB, D), row),
        pl.BlockSpec((BB, N, D), big),
        pl.BlockSpec((BB, N, D), big),
        pl.BlockSpec((BB, 3 * N), row),
        pl.BlockSpec((D, D), full2),
        pl.BlockSpec((D,), lambda i: (0,)),
        pl.BlockSpec((D, D), full2),
        pl.BlockSpec((D,), lambda i: (0,)),
        pl.BlockSpec((4, D), full2),
        pl.BlockSpec((4, 1), full2),
        pl.BlockSpec((2, D), full2),
        pl.BlockSpec((2, 1), full2),
    ]
    out_specs = [
        pl.BlockSpec((BB, N, D), big),
        pl.BlockSpec((BB, N, D), big),
        pl.BlockSpec((BB, 2 * N + D), row),
    ]
    return pl.pallas_call(
        _sbmem_kernel,
        grid=grid,
        in_specs=in_specs,
        out_specs=out_specs,
        out_shape=out_shapes,
        compiler_params=pltpu.CompilerParams(
            dimension_semantics=("arbitrary",)),
        interpret=interpret,
    )(hidden, working_keys, working_values, pua,
      Wk, bk, Wv, bv, Wg, bg, Wsx, bsx)


def kernel(hidden, working_keys, working_values, working_protection,
           working_usage, working_age, Wk, bk, Wv, bv, Wwg, bwg, Wmg, bmg,
           Wbg, bbg, Wig, big, Wso, bso, Wsp, bsp):
    Wg = jnp.concatenate([Wwg, Wmg, Wbg, Wig], axis=0)
    bg = jnp.stack([bwg, bmg, bbg, big], axis=0)
    Wsx = jnp.concatenate([Wso, Wsp], axis=0)
    bsx = jnp.stack([bso, bsp], axis=0)
    pua = jnp.concatenate(
        [working_protection, working_usage, working_age], axis=1)
    (uk, uv, po) = _run(
        hidden, working_keys, working_values, pua,
        Wk, bk, Wv, bv, Wg, bg, Wsx, bsx)
    uprot = po[:, 0:N]
    stats = {
        'write_strength': po[:, 2 * N + 0],
        'merge_preference': po[:, 2 * N + 1],
        'binding_strength': po[:, 2 * N + 2],
        'overwrite_ratio': po[:, 2 * N + 3],
        'protection_mean': po[:, 2 * N + 4],
        'max_similarity': po[:, 2 * N + 5],
        'slot_write_mass': po[:, N:2 * N],
    }
    return (uk, uv, uprot, stats)


# speedup vs baseline: 4.4024x; 1.0004x over previous
import functools

import jax
import jax.numpy as jnp
from jax.experimental import pallas as pl
from jax.experimental.pallas import tpu as pltpu

B, N, D = 1024, 256, 128
PROT_DECAY = 0.95
TEMP = 0.5
BB = 32
SB = 8

_bf16 = jnp.bfloat16
_f32 = jnp.float32


def _dot_t(a, b):
    return jax.lax.dot_general(
        a.astype(_bf16), b.astype(_bf16), (((1,), (1,)), ((), ())),
        preferred_element_type=_f32)


def _softmax_argmax(x, iota):
    m = jnp.max(x, axis=-1, keepdims=True)
    e = jnp.exp(x - m)
    s = jnp.sum(e, axis=-1, keepdims=True)
    d = e / s
    dmax = 1.0 / s
    return jnp.min(jnp.where(d == dmax, iota, N), axis=-1, keepdims=True)


def _sbmem_kernel(h_ref, wk_ref, wv_ref, pua_ref,
                  Wk_ref, bk_ref, Wv_ref, bv_ref,
                  Wg_ref, bg_ref, Wsx_ref, bsx_ref,
                  uk_ref, uv_ref, po_ref):
    prot_in = pua_ref[:, 0:N]
    usage_in = pua_ref[:, N:2 * N]
    age_in = pua_ref[:, 2 * N:3 * N]
    h = h_ref[...]
    ck = jnp.tanh(_dot_t(h, Wk_ref[...]) + bk_ref[...][None, :])
    cv = jnp.tanh(_dot_t(h, Wv_ref[...]) + bv_ref[...][None, :])

    gates = _dot_t(h, Wg_ref[...]) + bg_ref[...][:, 0][None, :]
    write_strength = jax.nn.sigmoid(gates[:, 0:1])
    g_mg = gates[:, 1:2]
    g_bg = gates[:, 2:3]
    importance = jax.nn.sigmoid(gates[:, 3:4])

    ck_norm = jnp.sqrt(jnp.sum(ck * ck, axis=-1, keepdims=True))
    ncand = ck / jnp.maximum(ck_norm, 1e-6)

    wk = wk_ref[...]
    key_norm = jnp.sqrt(jnp.sum(wk * wk, axis=-1))
    nkeys = wk / jnp.maximum(key_norm, 1e-6)[:, :, None]
    nkeys_bf = nkeys.astype(_bf16)
    ncand_bf = ncand.astype(_bf16)
    bsel = (jax.lax.broadcasted_iota(jnp.int32, (SB, SB, 1), 0) ==
            jax.lax.broadcasted_iota(jnp.int32, (SB, SB, 1), 1))
    sim_chunks = []
    for c in range(BB // SB):
        sa = jax.lax.dot_general(
            ncand_bf[c * SB:(c + 1) * SB],
            nkeys_bf[c * SB:(c + 1) * SB].reshape(SB * N, D),
            (((1,), (1,)), ((), ())),
            preferred_element_type=_f32).reshape(SB, SB, N)
        sim_chunks.append(jnp.sum(jnp.where(bsel, sa, 0.0), axis=1))
    sim = jnp.concatenate(sim_chunks, axis=0)

    wv = wv_ref[...]
    val_norm = jnp.sqrt(jnp.sum(wv * wv, axis=-1))
    norm_occ = jnp.clip(val_norm / (D ** 0.5), 0.0, 1.0)
    sx = jax.lax.dot_general(
        Wsx_ref[...].astype(_bf16), wv.astype(_bf16).reshape(BB * N, D),
        (((1,), (1,)), ((), ())),
        preferred_element_type=_f32).reshape(2, BB, N)
    learned_occ = jax.nn.sigmoid(sx[0] + bsx_ref[0, 0])
    learned_prot = jax.nn.sigmoid(sx[1] + bsx_ref[1, 0])
    occupancy = jnp.clip(0.5 * learned_occ + 0.5 * norm_occ, 0.0, 1.0)
    eff_prot = jnp.clip(0.4 * learned_prot + 0.6 * prot_in, 0.0, 1.0)
    eff_usage = jnp.clip(0.5 * occupancy + 0.5 * usage_in, 0.0, 1.0)
    eff_age = jnp.clip(age_in, 0.0, 1.0)

    replace_scores = (1.15 * (1.0 - occupancy) + 0.85 * (1.0 - eff_prot)
                      + 0.65 * eff_age + 0.45 * (1.0 - eff_usage)
                      + 0.25 * (1.0 - sim))

    iota = jax.lax.broadcasted_iota(jnp.int32, (BB, N), 1)
    merge_index = _softmax_argmax(sim / TEMP, iota)
    replace_index = _softmax_argmax(replace_scores / TEMP, iota)

    merge_oh = (iota == merge_index).astype(_f32)
    max_sim = jnp.sum(sim * merge_oh, axis=-1, keepdims=True)
    matched_occ = jnp.sum(occupancy * merge_oh, axis=-1, keepdims=True)
    matched_usage = jnp.sum(eff_usage * merge_oh, axis=-1, keepdims=True)
    matched_age = jnp.sum(eff_age * merge_oh, axis=-1, keepdims=True)

    merge_pref = jax.nn.sigmoid(g_mg + 2.4 * max_sim
                                + 1.6 * (matched_occ - 0.5)
                                + 1.0 * (matched_usage - 0.5)
                                - 0.8 * matched_age)
    merge_cand = (max_sim > 0.55) & (matched_occ > 0.35)
    use_merge = (merge_pref >= 0.5) & merge_cand
    target_index = jnp.where(use_merge, merge_index, replace_index)
    target_oh = (iota == target_index).astype(_f32)

    binding = jax.nn.sigmoid(g_bg + 2.2 * max_sim)
    conflict = jnp.clip(1.0 - sim, 0.0, 1.0)
    overwrite = ((0.15 + 0.85 * write_strength) * target_oh
                 * (1.0 - 0.65 * eff_prot * conflict))

    key_mix = jnp.where(use_merge, 0.22 + 0.38 * binding, 0.78 + 0.18 * binding)
    value_mix = jnp.where(use_merge, 0.45 + 0.35 * importance, 0.75 + 0.2 * importance)

    owk = (overwrite * key_mix)[:, :, None]
    owv = (overwrite * value_mix)[:, :, None]
    uk_ref[...] = wk + owk * (ck[:, None, :] - wk)
    uv_ref[...] = wv + owv * (cv[:, None, :] - wv)

    prot_boost = overwrite * (0.5 + 0.5 * importance)
    uprot = jnp.clip(prot_in * PROT_DECAY + prot_boost, 0.0, 1.0)
    stcat = jnp.concatenate(
        [write_strength, merge_pref, binding,
         jnp.mean(overwrite, axis=-1, keepdims=True),
         jnp.mean(uprot, axis=-1, keepdims=True), max_sim,
         jnp.zeros((BB, D - 6), _f32)], axis=1)
    po_ref[...] = jnp.concatenate([uprot, overwrite, stcat], axis=1)


@functools.partial(jax.jit, static_argnames=("interpret",))
def _run(hidden, working_keys, working_values, pua,
         Wk, bk, Wv, bv, Wg, bg, Wsx, bsx,
         interpret=False):
    grid = (B // BB,)
    big = lambda i: (i, 0, 0)
    row = lambda i: (i, 0)
    full2 = lambda i: (0, 0)
    out_shapes = [
        jax.ShapeDtypeStruct((B, N, D), jnp.float32),
        jax.ShapeDtypeStruct((B, N, D), jnp.float32),
        jax.ShapeDtypeStruct((B, 2 * N + D), jnp.float32),
    ]
    in_specs = [
        pl.BlockSpec((BB, D), row),
        pl.BlockSpec((BB, N, D), big),
        pl.BlockSpec((BB, N, D), big),
        pl.BlockSpec((BB, 3 * N), row),
        pl.BlockSpec((D, D), full2),
        pl.BlockSpec((D,), lambda i: (0,)),
        pl.BlockSpec((D, D), full2),
        pl.BlockSpec((D,), lambda i: (0,)),
        pl.BlockSpec((4, D), full2),
        pl.BlockSpec((4, 1), full2),
        pl.BlockSpec((2, D), full2),
        pl.BlockSpec((2, 1), full2),
    ]
    out_specs = [
        pl.BlockSpec((BB, N, D), big),
        pl.BlockSpec((BB, N, D), big),
        pl.BlockSpec((BB, 2 * N + D), row),
    ]
    return pl.pallas_call(
        _sbmem_kernel,
        grid=grid,
        in_specs=in_specs,
        out_specs=out_specs,
        out_shape=out_shapes,
        compiler_params=pltpu.CompilerParams(
            dimension_semantics=("parallel",)),
        interpret=interpret,
    )(hidden, working_keys, working_values, pua,
      Wk, bk, Wv, bv, Wg, bg, Wsx, bsx)


def kernel(hidden, working_keys, working_values, working_protection,
           working_usage, working_age, Wk, bk, Wv, bv, Wwg, bwg, Wmg, bmg,
           Wbg, bbg, Wig, big, Wso, bso, Wsp, bsp):
    Wg = jnp.concatenate([Wwg, Wmg, Wbg, Wig], axis=0)
    bg = jnp.stack([bwg, bmg, bbg, big], axis=0)
    Wsx = jnp.concatenate([Wso, Wsp], axis=0)
    bsx = jnp.stack([bso, bsp], axis=0)
    pua = jnp.concatenate(
        [working_protection, working_usage, working_age], axis=1)
    (uk, uv, po) = _run(
        hidden, working_keys, working_values, pua,
        Wk, bk, Wv, bv, Wg, bg, Wsx, bsx)
    uprot = po[:, 0:N]
    stats = {
        'write_strength': po[:, 2 * N + 0],
        'merge_preference': po[:, 2 * N + 1],
        'binding_strength': po[:, 2 * N + 2],
        'overwrite_ratio': po[:, 2 * N + 3],
        'protection_mean': po[:, 2 * N + 4],
        'max_similarity': po[:, 2 * N + 5],
        'slot_write_mass': po[:, N:2 * N],
    }
    return (uk, uv, uprot, stats)
